# baseline (speedup 1.0000x reference)
import functools

import jax
import jax.numpy as jnp
from jax import lax
from jax.experimental import pallas as pl
from jax.experimental.pallas import tpu as pltpu
from jax.experimental.pallas import tpu_sc as plsc

B = 16384
D = 32
NC = 2
NS = 16
NW = NC * NS
BPW = B // NW
CH = 128
NCH = BPW // CH


def _gather_body(iidx_hbm, uidx_hbm, item_tab, user_tab, ilv_hbm, ulv_hbm,
                 iidx_v, uidx_v, irows_v, urows_v, sem):
    wid = lax.axis_index("s") * NC + lax.axis_index("c")
    pltpu.sync_copy(iidx_hbm.at[pl.ds(wid * NCH, NCH)], iidx_v)
    pltpu.sync_copy(uidx_hbm.at[pl.ds(wid * NCH, NCH)], uidx_v)
    copies = []
    for j in range(NCH):
        copies.append(pltpu.async_copy(
            item_tab.at[iidx_v.at[j]], irows_v.at[pl.ds(j * CH, CH)], sem))
        copies.append(pltpu.async_copy(
            user_tab.at[uidx_v.at[j]], urows_v.at[pl.ds(j * CH, CH)], sem))
    for c in copies:
        c.wait()
    pltpu.sync_copy(irows_v, ilv_hbm.at[pl.ds(wid * BPW, BPW)])
    pltpu.sync_copy(urows_v, ulv_hbm.at[pl.ds(wid * BPW, BPW)])


_gather = pl.kernel(
    _gather_body,
    mesh=plsc.VectorSubcoreMesh(core_axis_name="c", subcore_axis_name="s"),
    out_type=[
        jax.ShapeDtypeStruct((B, D), jnp.float32),
        jax.ShapeDtypeStruct((B, D), jnp.float32),
    ],
    scratch_types=[
        pltpu.VMEM((NCH, CH), jnp.int32),
        pltpu.VMEM((NCH, CH), jnp.int32),
        pltpu.VMEM((BPW, D), jnp.float32),
        pltpu.VMEM((BPW, D), jnp.float32),
        pltpu.SemaphoreType.DMA,
    ],
    compiler_params=pltpu.CompilerParams(use_tc_tiling_on_sc=False),
)


def _mlp_body(ilv_ref, ulv_ref, wt_ref, b_ref, out_ref):
    dp = ilv_ref[...] * ulv_ref[...]
    acc = jnp.dot(dp, wt_ref[...], preferred_element_type=jnp.float32)
    out_ref[...] = jax.nn.sigmoid(acc + b_ref[...])


_BB = 2048


_mlp = pl.pallas_call(
    _mlp_body,
    grid=(B // _BB,),
    in_specs=[
        pl.BlockSpec((_BB, D), lambda i: (i, 0)),
        pl.BlockSpec((_BB, D), lambda i: (i, 0)),
        pl.BlockSpec((D, D), lambda i: (0, 0)),
        pl.BlockSpec((1, D), lambda i: (0, 0)),
    ],
    out_specs=pl.BlockSpec((_BB, D), lambda i: (i, 0)),
    out_shape=jax.ShapeDtypeStruct((B, D), jnp.float32),
)


def kernel(item_vec, user_vec, item_table, user_table, W, b):
    iidx = item_vec.astype(jnp.int32).reshape(NW * NCH, CH)
    uidx = user_vec.astype(jnp.int32).reshape(NW * NCH, CH)
    ilv, ulv = _gather(iidx, uidx, item_table, user_table)
    return _mlp(ilv, ulv, W.T, b.reshape(1, D))
